# baseline (speedup 1.0000x reference)
import numpy as np

import jax
import jax.numpy as jnp
from jax.experimental import pallas as pl
from jax.experimental.pallas import tpu as pltpu

N = 10000
IN_DIM = 128
H1 = 128
H2 = 64

ENC_R = 400
NSUB = 10
SUB = ENC_R // NSUB
N_BLK = N // ENC_R
COL_W = 1280
N_COL = -(-N // COL_W)
N_PAD = N_COL * COL_W
_COL_READY = [min(-(-(COL_W * (c + 1)) // ENC_R) - 1, N_BLK - 1)
              for c in range(N_COL)]


def _build_schedule():
    act, kk, ti, tj = [], [], [], []
    li, lj = 0, 0
    for k in range(N_BLK):
        act.append(0); kk.append(k); ti.append(li); tj.append(lj)
        tiles = []
        for c in range(N_COL):
            if _COL_READY[c] == k:
                tiles += [(i, c) for i in range(k + 1)]
        tiles += [(k, c) for c in range(N_COL) if _COL_READY[c] < k]
        for (i, c) in tiles:
            act.append(1); kk.append(k); ti.append(i); tj.append(c)
            li, lj = i, c
    return (np.array(act, np.int32), np.array(kk, np.int32),
            np.array(ti, np.int32), np.array(tj, np.int32))


_ACT, _KK, _TI, _TJ = _build_schedule()
N_STEPS = len(_ACT)


def _kernel(act_ref, k_ref, ti_ref, tj_ref,
            g_hbm, f_hbm, w1_ref, lns_ref, lnb_ref, w2_ref, b2_ref,
            w3_ref, b3_ref, eps_ref,
            mu_ref, logvar_ref, z_ref, adj_ref,
            gbuf, f_vmem, sup_ref, zs_ref, zt_ref, fsem, sems):
    s = pl.program_id(0)
    act = act_ref[s]
    k = k_ref[s]

    def g_copy(blk, slot, j):
        return pltpu.make_async_copy(
            g_hbm.at[pl.ds(blk * ENC_R + j * SUB, SUB), :],
            gbuf.at[slot, pl.ds(j * SUB, SUB), :],
            sems.at[slot, j])

    @pl.when(s == 0)
    def _():
        for j in range(NSUB):
            g_copy(0, 0, j).start()
        cp = pltpu.make_async_copy(f_hbm, f_vmem, fsem)
        cp.start()
        cp.wait()
        sup_ref[...] = jnp.dot(f_vmem[...], w1_ref[...],
                               preferred_element_type=jnp.float32
                               ).astype(jnp.bfloat16)
        zs_ref[pl.ds(N, N_PAD - N), :] = jnp.zeros((N_PAD - N, H2),
                                                   jnp.float32)

    @pl.when(act == 0)
    def _enc():
        slot = jax.lax.rem(k, 2)

        @pl.when(k + 1 < N_BLK)
        def _():
            for j in range(NSUB):
                g_copy(k + 1, 1 - slot, j).start()

        for j in range(NSUB):
            g_copy(k, slot, j).wait()

        h1 = jnp.concatenate(
            [jnp.dot(gbuf[slot, j * SUB:(j + 1) * SUB, :].astype(jnp.bfloat16),
                     sup_ref[...], preferred_element_type=jnp.float32)
             for j in range(NSUB)],
            axis=0)
        h1 = jnp.maximum(h1, 0.0)
        mean = jnp.mean(h1, axis=-1, keepdims=True)
        var = jnp.mean((h1 - mean) ** 2, axis=-1, keepdims=True)
        h = (h1 - mean) / jnp.sqrt(var + 1e-5) * lns_ref[...] + lnb_ref[...]
        mu = jnp.dot(h, w2_ref[...], preferred_element_type=jnp.float32) + b2_ref[...]
        logvar = jnp.dot(h, w3_ref[...], preferred_element_type=jnp.float32) + b3_ref[...]
        z = eps_ref[...] * jnp.exp(logvar) + mu
        mu_ref[...] = mu
        logvar_ref[...] = logvar
        z_ref[...] = z
        zs_ref[pl.ds(k * ENC_R, ENC_R), :] = z

        for c in range(N_COL):
            @pl.when(k == _COL_READY[c])
            def _(c=c):
                zt_ref[c] = (zs_ref[pl.ds(c * COL_W, COL_W), :]
                             .T.astype(jnp.bfloat16))

    @pl.when(act == 1)
    def _tile():
        i = ti_ref[s]
        col = tj_ref[s]
        lhs = zs_ref[pl.ds(i * ENC_R, ENC_R), :].astype(jnp.bfloat16)
        adj_ref[...] = jnp.dot(lhs, zt_ref[col],
                               preferred_element_type=jnp.float32)


@jax.jit
def kernel(g, features, W1, ln_scale, ln_bias, W2, b2, W3, b3):
    eps = jax.random.normal(jax.random.key(42), (N, H2), dtype=jnp.float32)
    lns = ln_scale.reshape(1, H1)
    lnb = ln_bias.reshape(1, H1)
    b2r = b2.reshape(1, H2)
    b3r = b3.reshape(1, H2)

    grid_spec = pltpu.PrefetchScalarGridSpec(
        num_scalar_prefetch=4,
        grid=(N_STEPS,),
        in_specs=[
            pl.BlockSpec(memory_space=pltpu.MemorySpace.HBM),
            pl.BlockSpec(memory_space=pltpu.MemorySpace.HBM),
            pl.BlockSpec((IN_DIM, H1), lambda s, a, kk, ti, tj: (0, 0)),
            pl.BlockSpec((1, H1), lambda s, a, kk, ti, tj: (0, 0)),
            pl.BlockSpec((1, H1), lambda s, a, kk, ti, tj: (0, 0)),
            pl.BlockSpec((H1, H2), lambda s, a, kk, ti, tj: (0, 0)),
            pl.BlockSpec((1, H2), lambda s, a, kk, ti, tj: (0, 0)),
            pl.BlockSpec((H1, H2), lambda s, a, kk, ti, tj: (0, 0)),
            pl.BlockSpec((1, H2), lambda s, a, kk, ti, tj: (0, 0)),
            pl.BlockSpec((ENC_R, H2), lambda s, a, kk, ti, tj: (kk[s], 0)),
        ],
        out_specs=[
            pl.BlockSpec((ENC_R, H2), lambda s, a, kk, ti, tj: (kk[s], 0)),
            pl.BlockSpec((ENC_R, H2), lambda s, a, kk, ti, tj: (kk[s], 0)),
            pl.BlockSpec((ENC_R, H2), lambda s, a, kk, ti, tj: (kk[s], 0)),
            pl.BlockSpec((ENC_R, COL_W),
                         lambda s, a, kk, ti, tj: (ti[s], tj[s])),
        ],
        scratch_shapes=[
            pltpu.VMEM((2, ENC_R, N), jnp.float32),
            pltpu.VMEM((N, IN_DIM), jnp.float32),
            pltpu.VMEM((N, H1), jnp.bfloat16),
            pltpu.VMEM((N_PAD, H2), jnp.float32),
            pltpu.VMEM((N_COL, H2, COL_W), jnp.bfloat16),
            pltpu.SemaphoreType.DMA,
            pltpu.SemaphoreType.DMA((2, NSUB)),
        ],
    )

    mu, logvar, z, adj = pl.pallas_call(
        _kernel,
        grid_spec=grid_spec,
        out_shape=[
            jax.ShapeDtypeStruct((N, H2), jnp.float32),
            jax.ShapeDtypeStruct((N, H2), jnp.float32),
            jax.ShapeDtypeStruct((N, H2), jnp.float32),
            jax.ShapeDtypeStruct((N, N), jnp.float32),
        ],
    )(jnp.asarray(_ACT), jnp.asarray(_KK), jnp.asarray(_TI), jnp.asarray(_TJ),
      g, features, W1, lns, lnb, W2, b2r, W3, b3r, eps)

    return (adj, mu, logvar, z)


# speedup vs baseline: 1.1764x; 1.1764x over previous
import jax
import jax.numpy as jnp
from jax.experimental import pallas as pl
from jax.experimental.pallas import tpu as pltpu

N = 10000
IN_DIM = 128
H1 = 128
H2 = 64

ENC_R = 400
NSUB = 10
SUB = ENC_R // NSUB
DEC_R = 400


def _enc_kernel(g_hbm, f_hbm, w1_ref, lns_ref, lnb_ref, w2_ref, b2_ref,
                w3_ref, b3_ref, eps_ref, mu_ref, logvar_ref, z_ref,
                gbuf0, gbuf1, gbuf2, gbuf3, gbuf4, f_vmem, sup_ref,
                fsem, sems):
    i = pl.program_id(0)
    nsteps = pl.num_programs(0)
    gbufs = (gbuf0, gbuf1, gbuf2, gbuf3, gbuf4)

    def g_copy(step, slot, j):
        return pltpu.make_async_copy(
            g_hbm.at[pl.ds(step * ENC_R + j * SUB, SUB), :],
            gbufs[j % 5].at[slot, pl.ds((j // 5) * SUB, SUB), :],
            sems.at[slot, j])

    slot = jax.lax.rem(i, 2)

    @pl.when(i == 0)
    def _():
        for j in range(NSUB):
            g_copy(0, 0, j).start()
        cp = pltpu.make_async_copy(f_hbm, f_vmem, fsem)
        cp.start()
        cp.wait()
        sup_ref[...] = jnp.dot(f_vmem[...], w1_ref[...],
                               preferred_element_type=jnp.float32
                               ).astype(jnp.bfloat16)

    @pl.when(i + 1 < nsteps)
    def _():
        for j in range(NSUB):
            g_copy(i + 1, 1 - slot, j).start()

    for j in range(NSUB):
        g_copy(i, slot, j).wait()

    h1 = jnp.concatenate(
        [jnp.dot(gbufs[j % 5][slot, (j // 5) * SUB:(j // 5 + 1) * SUB, :]
                 .astype(jnp.bfloat16),
                 sup_ref[...], preferred_element_type=jnp.float32)
         for j in range(NSUB)],
        axis=0)
    h1 = jnp.maximum(h1, 0.0)
    mean = jnp.mean(h1, axis=-1, keepdims=True)
    var = jnp.mean((h1 - mean) ** 2, axis=-1, keepdims=True)
    h = (h1 - mean) / jnp.sqrt(var + 1e-5) * lns_ref[...] + lnb_ref[...]
    mu = jnp.dot(h, w2_ref[...], preferred_element_type=jnp.float32) + b2_ref[...]
    logvar = jnp.dot(h, w3_ref[...], preferred_element_type=jnp.float32) + b3_ref[...]
    z = eps_ref[...] * jnp.exp(logvar) + mu
    mu_ref[...] = mu
    logvar_ref[...] = logvar
    z_ref[...] = z


def _dec_kernel(zi_ref, zt_hbm, adj_ref, zt_vmem, sem):
    i = pl.program_id(0)

    @pl.when(i == 0)
    def _():
        cp = pltpu.make_async_copy(zt_hbm, zt_vmem, sem)
        cp.start()
        cp.wait()

    adj_ref[...] = jnp.dot(zi_ref[...], zt_vmem[...],
                           preferred_element_type=jnp.float32)


@jax.jit
def kernel(g, features, W1, ln_scale, ln_bias, W2, b2, W3, b3):
    eps = jax.random.normal(jax.random.key(42), (N, H2), dtype=jnp.float32)
    lns = ln_scale.reshape(1, H1)
    lnb = ln_bias.reshape(1, H1)
    b2r = b2.reshape(1, H2)
    b3r = b3.reshape(1, H2)

    mu, logvar, z = pl.pallas_call(
        _enc_kernel,
        grid=(N // ENC_R,),
        in_specs=[
            pl.BlockSpec(memory_space=pltpu.MemorySpace.HBM),
            pl.BlockSpec(memory_space=pltpu.MemorySpace.HBM),
            pl.BlockSpec((IN_DIM, H1), lambda i: (0, 0)),
            pl.BlockSpec((1, H1), lambda i: (0, 0)),
            pl.BlockSpec((1, H1), lambda i: (0, 0)),
            pl.BlockSpec((H1, H2), lambda i: (0, 0)),
            pl.BlockSpec((1, H2), lambda i: (0, 0)),
            pl.BlockSpec((H1, H2), lambda i: (0, 0)),
            pl.BlockSpec((1, H2), lambda i: (0, 0)),
            pl.BlockSpec((ENC_R, H2), lambda i: (i, 0)),
        ],
        out_specs=[
            pl.BlockSpec((ENC_R, H2), lambda i: (i, 0)),
            pl.BlockSpec((ENC_R, H2), lambda i: (i, 0)),
            pl.BlockSpec((ENC_R, H2), lambda i: (i, 0)),
        ],
        out_shape=[
            jax.ShapeDtypeStruct((N, H2), jnp.float32),
            jax.ShapeDtypeStruct((N, H2), jnp.float32),
            jax.ShapeDtypeStruct((N, H2), jnp.float32),
        ],
        scratch_shapes=[
            pltpu.VMEM((2, 2 * SUB, N), jnp.float32),
            pltpu.VMEM((2, 2 * SUB, N), jnp.float32),
            pltpu.VMEM((2, 2 * SUB, N), jnp.float32),
            pltpu.VMEM((2, 2 * SUB, N), jnp.float32),
            pltpu.VMEM((2, 2 * SUB, N), jnp.float32),
            pltpu.VMEM((N, IN_DIM), jnp.float32),
            pltpu.VMEM((N, H1), jnp.bfloat16),
            pltpu.SemaphoreType.DMA,
            pltpu.SemaphoreType.DMA((2, NSUB)),
        ],
    )(g, features, W1, lns, lnb, W2, b2r, W3, b3r, eps)

    zb = z.astype(jnp.bfloat16)
    ztb = zb.T

    adj = pl.pallas_call(
        _dec_kernel,
        grid=(N // DEC_R,),
        in_specs=[
            pl.BlockSpec((DEC_R, H2), lambda i: (i, 0)),
            pl.BlockSpec(memory_space=pltpu.MemorySpace.HBM),
        ],
        out_specs=pl.BlockSpec((DEC_R, N), lambda i: (i, 0)),
        out_shape=jax.ShapeDtypeStruct((N, N), jnp.float32),
        scratch_shapes=[
            pltpu.VMEM((H2, N), jnp.bfloat16),
            pltpu.SemaphoreType.DMA,
        ],
    )(zb, ztb)

    return (adj, mu, logvar, z)


# speedup vs baseline: 1.1805x; 1.0035x over previous
import jax
import jax.numpy as jnp
from jax.experimental import pallas as pl
from jax.experimental.pallas import tpu as pltpu

N = 10000
IN_DIM = 128
H1 = 128
H2 = 64

ENC_R = 400
DEC_R = 400


def _enc_kernel(ga_ref, gb_ref, f_hbm, w1_ref, lns_ref, lnb_ref, w2_ref,
                b2_ref, w3_ref, b3_ref, eps_ref, mu_ref, logvar_ref, z_ref,
                f_vmem, sup_ref, sem):
    i = pl.program_id(0)

    @pl.when(i == 0)
    def _():
        cp = pltpu.make_async_copy(f_hbm, f_vmem, sem)
        cp.start()
        cp.wait()
        sup_ref[...] = jnp.dot(f_vmem[...], w1_ref[...],
                               preferred_element_type=jnp.float32
                               ).astype(jnp.bfloat16)

    g_blk = jnp.concatenate([ga_ref[...], gb_ref[...]], axis=0)
    h1 = jnp.dot(g_blk.astype(jnp.bfloat16), sup_ref[...],
                 preferred_element_type=jnp.float32)
    h1 = jnp.maximum(h1, 0.0)
    mean = jnp.mean(h1, axis=-1, keepdims=True)
    var = jnp.mean((h1 - mean) ** 2, axis=-1, keepdims=True)
    h = (h1 - mean) / jnp.sqrt(var + 1e-5) * lns_ref[...] + lnb_ref[...]
    mu = jnp.dot(h, w2_ref[...], preferred_element_type=jnp.float32) + b2_ref[...]
    logvar = jnp.dot(h, w3_ref[...], preferred_element_type=jnp.float32) + b3_ref[...]
    z = eps_ref[...] * jnp.exp(logvar) + mu
    mu_ref[...] = mu
    logvar_ref[...] = logvar
    z_ref[...] = z


def _dec_kernel(zi_ref, zt_hbm, adj_ref, zt_vmem, sem):
    i = pl.program_id(0)

    @pl.when(i == 0)
    def _():
        cp = pltpu.make_async_copy(zt_hbm, zt_vmem, sem)
        cp.start()
        cp.wait()

    adj_ref[...] = jnp.dot(zi_ref[...], zt_vmem[...],
                           preferred_element_type=jnp.float32)


@jax.jit
def kernel(g, features, W1, ln_scale, ln_bias, W2, b2, W3, b3):
    eps = jax.random.normal(jax.random.key(42), (N, H2), dtype=jnp.float32)
    lns = ln_scale.reshape(1, H1)
    lnb = ln_bias.reshape(1, H1)
    b2r = b2.reshape(1, H2)
    b3r = b3.reshape(1, H2)

    mu, logvar, z = pl.pallas_call(
        _enc_kernel,
        grid=(N // ENC_R,),
        in_specs=[
            pl.BlockSpec((ENC_R // 2, N), lambda i: (2 * i, 0)),
            pl.BlockSpec((ENC_R // 2, N), lambda i: (2 * i + 1, 0)),
            pl.BlockSpec(memory_space=pltpu.MemorySpace.HBM),
            pl.BlockSpec((IN_DIM, H1), lambda i: (0, 0)),
            pl.BlockSpec((1, H1), lambda i: (0, 0)),
            pl.BlockSpec((1, H1), lambda i: (0, 0)),
            pl.BlockSpec((H1, H2), lambda i: (0, 0)),
            pl.BlockSpec((1, H2), lambda i: (0, 0)),
            pl.BlockSpec((H1, H2), lambda i: (0, 0)),
            pl.BlockSpec((1, H2), lambda i: (0, 0)),
            pl.BlockSpec((ENC_R, H2), lambda i: (i, 0)),
        ],
        out_specs=[
            pl.BlockSpec((ENC_R, H2), lambda i: (i, 0)),
            pl.BlockSpec((ENC_R, H2), lambda i: (i, 0)),
            pl.BlockSpec((ENC_R, H2), lambda i: (i, 0)),
        ],
        out_shape=[
            jax.ShapeDtypeStruct((N, H2), jnp.float32),
            jax.ShapeDtypeStruct((N, H2), jnp.float32),
            jax.ShapeDtypeStruct((N, H2), jnp.float32),
        ],
        scratch_shapes=[
            pltpu.VMEM((N, IN_DIM), jnp.float32),
            pltpu.VMEM((N, H1), jnp.bfloat16),
            pltpu.SemaphoreType.DMA,
        ],
    )(g, g, features, W1, lns, lnb, W2, b2r, W3, b3r, eps)

    zb = z.astype(jnp.bfloat16)
    ztb = zb.T

    adj = pl.pallas_call(
        _dec_kernel,
        grid=(N // DEC_R,),
        in_specs=[
            pl.BlockSpec((DEC_R, H2), lambda i: (i, 0)),
            pl.BlockSpec(memory_space=pltpu.MemorySpace.HBM),
        ],
        out_specs=pl.BlockSpec((DEC_R, N), lambda i: (i, 0)),
        out_shape=jax.ShapeDtypeStruct((N, N), jnp.float32),
        scratch_shapes=[
            pltpu.VMEM((H2, N), jnp.bfloat16),
            pltpu.SemaphoreType.DMA,
        ],
    )(zb, ztb)

    return (adj, mu, logvar, z)
